# baseline (speedup 1.0000x reference)
import functools

import jax
import jax.numpy as jnp
from jax import lax
from jax.experimental import pallas as pl
from jax.experimental.pallas import tpu as pltpu
from jax.experimental.pallas import tpu_sc as plsc

N = 10000
D = 256
K = 3
E = 160000

HALF = 128
NC = 2
NS = 16
L = 16
C = 96
NCH = 106
EPT = NCH * C
EPAD = EPT * NS
NPAD = 10240
RPT = NPAD // NS
CW = 16
PAD_DST = NPAD - 8


@functools.cache
def _get_sc_segment():
  mesh = plsc.VectorSubcoreMesh(core_axis_name="c", subcore_axis_name="s",
                                num_cores=NC, num_subcores=NS)

  @functools.partial(
      pl.kernel,
      out_type=(
          jax.ShapeDtypeStruct((NC, K, NPAD, HALF), jnp.float32),
          jax.ShapeDtypeStruct((NC, K, NPAD, CW), jnp.float32),
      ),
      mesh=mesh,
      compiler_params=pltpu.CompilerParams(use_tc_tiling_on_sc=False,
                                           needs_layout_passes=False),
      scratch_types=[
          pltpu.VMEM((2, C), jnp.int32),
          pltpu.VMEM((2, C), jnp.int32),
          pltpu.VMEM((2, C, HALF), jnp.bfloat16),
          pltpu.VMEM((2, C, HALF), jnp.float32),
          pltpu.VMEM((C, CW), jnp.float32),
          pltpu.VMEM_SHARED((NPAD, HALF), jnp.float32),
          pltpu.VMEM_SHARED((NPAD, CW), jnp.float32),
          pltpu.SemaphoreType.DMA,
          pltpu.SemaphoreType.DMA,
          pltpu.SemaphoreType.DMA,
          pltpu.SemaphoreType.DMA,
      ],
  )
  def _sc_segment(edges_hbm, x2_hbm, z128_hbm, z16_hbm, ones_hbm,
                  s_out, cnt_out,
                  gidx_v, dst_v, rows_bf, rows_f, ones_v,
                  shared, cshared, gsem0, gsem1, ssem0, ssem1):
    cid = lax.axis_index("c")
    sid = lax.axis_index("s")
    r0 = pl.multiple_of(sid * RPT, 8)
    gsems = (gsem0, gsem1)
    ssems = (ssem0, ssem1)

    pltpu.sync_copy(ones_hbm, ones_v)

    for k in range(K):
      kbase = k * N
      src_base = (2 * k) * EPAD
      dst_base = (2 * k + 1) * EPAD

      pltpu.sync_copy(z128_hbm.at[pl.ds(r0, RPT)], shared.at[pl.ds(r0, RPT)])
      pltpu.sync_copy(z16_hbm.at[pl.ds(r0, RPT)], cshared.at[pl.ds(r0, RPT)])

      plsc.subcore_barrier()

      def fire(g, b):
        e0 = pl.multiple_of(sid * EPT + g * C, 8)
        pltpu.sync_copy(edges_hbm.at[pl.ds(src_base + e0, C)], gidx_v.at[b])
        pltpu.sync_copy(edges_hbm.at[pl.ds(dst_base + e0, C)], dst_v.at[b])
        for j in range(C // L):
          s16 = gidx_v[b, pl.ds(j * L, L)]
          gidx_v[b, pl.ds(j * L, L)] = (s16 + kbase) * 2 + cid
        pltpu.async_copy(x2_hbm.at[gidx_v.at[b]], rows_bf.at[b], gsems[b])

      fire(0, 0)
      fire(1, 1)

      def body(i2, carry):
        for b in range(2):
          g = i2 * 2 + b
          pltpu.make_async_copy(x2_hbm.at[gidx_v.at[b]], rows_bf.at[b],
                                gsems[b]).wait()

          def conv_row(r, carry2):
            for j in range(HALF // 32):
              v = rows_bf[b, r, pl.ds(j * 32, 32)]
              lo, hi = plsc.unpack(v, format=plsc.PackFormat.INTERLEAVED)
              rows_f[b, r, pl.ds(j * 32, L)] = lo
              rows_f[b, r, pl.ds(j * 32 + L, L)] = hi
            return carry2

          lax.fori_loop(0, C, conv_row, 0)

          pltpu.async_copy(rows_f.at[b], shared.at[dst_v.at[b]], ssems[b],
                           add=True)

          @pl.when(cid == b)
          def _():
            pltpu.sync_copy(ones_v, cshared.at[dst_v.at[b]], add=True)

          @pl.when(i2 < NCH // 2 - 1)
          def _():
            pltpu.make_async_copy(rows_f.at[b], shared.at[dst_v.at[b]],
                                  ssems[b]).wait()
            fire(g + 2, b)

        return carry

      lax.fori_loop(0, NCH // 2, body, 0)
      for b in range(2):
        pltpu.make_async_copy(rows_f.at[b], shared.at[dst_v.at[b]],
                              ssems[b]).wait()
      plsc.subcore_barrier()

      pltpu.sync_copy(shared.at[pl.ds(r0, RPT)],
                      s_out.at[cid, k, pl.ds(r0, RPT)])
      pltpu.sync_copy(cshared.at[pl.ds(r0, RPT)],
                      cnt_out.at[cid, k, pl.ds(r0, RPT)])

      plsc.subcore_barrier()

  return _sc_segment


def _prep_body(wl_ref, bl_ref, wr_ref, wk_ref, a_ref, b_ref, c_ref):
    for k in range(K):
        top = wk_ref[k, :D, :]
        bot = wk_ref[k, D:, :]
        a_ref[k] = top + jnp.dot(wr_ref[...], bot,
                                 preferred_element_type=jnp.float32)
        b_ref[k] = jnp.dot(wl_ref[...], bot,
                           preferred_element_type=jnp.float32)
    bsum = wk_ref[0, D:, :] + wk_ref[1, D:, :] + wk_ref[2, D:, :]
    c_ref[...] = jnp.dot(bl_ref[...], bsum,
                         preferred_element_type=jnp.float32)


_prep_weights = pl.pallas_call(
    _prep_body,
    out_shape=(
        jax.ShapeDtypeStruct((K, D, D), jnp.float32),
        jax.ShapeDtypeStruct((K, D, D), jnp.float32),
        jax.ShapeDtypeStruct((1, D), jnp.float32),
    ),
)

R_BLK = 2000


def _dense_body(x_ref, sl_ref, sr_ref, c0_ref, c1_ref, a_ref, b_ref, c_ref,
                o_ref):
    acc = jnp.broadcast_to(c_ref[...], (R_BLK, D))
    for k in range(K):
        cnt = c0_ref[k][:, 0:1] + c1_ref[k][:, 0:1]
        m = jnp.maximum(cnt, 1.0)
        agg_l = sl_ref[k] / m
        agg_r = sr_ref[k] / m
        acc = acc + jnp.dot(x_ref[k], a_ref[k],
                            preferred_element_type=jnp.float32)
        acc = acc + jnp.dot(agg_l, b_ref[k, :HALF, :],
                            preferred_element_type=jnp.float32)
        acc = acc + jnp.dot(agg_r, b_ref[k, HALF:, :],
                            preferred_element_type=jnp.float32)
    h = jnp.maximum(acc, 0.0)
    norm = jnp.sqrt(jnp.sum(h * h, axis=1, keepdims=True))
    o_ref[...] = h / jnp.maximum(norm, 1e-12)


_dense = pl.pallas_call(
    _dense_body,
    grid=(N // R_BLK,),
    in_specs=[
        pl.BlockSpec((K, R_BLK, D), lambda i: (0, i, 0)),
        pl.BlockSpec((K, R_BLK, HALF), lambda i: (0, i, 0)),
        pl.BlockSpec((K, R_BLK, HALF), lambda i: (0, i, 0)),
        pl.BlockSpec((K, R_BLK, CW), lambda i: (0, i, 0)),
        pl.BlockSpec((K, R_BLK, CW), lambda i: (0, i, 0)),
        pl.BlockSpec((K, D, D), lambda i: (0, 0, 0)),
        pl.BlockSpec((K, D, D), lambda i: (0, 0, 0)),
        pl.BlockSpec((1, D), lambda i: (0, 0)),
    ],
    out_specs=pl.BlockSpec((R_BLK, D), lambda i: (i, 0)),
    out_shape=jax.ShapeDtypeStruct((N, D), jnp.float32),
)


@jax.jit
def kernel(H_K_prev, edgelists, W_l, b_l, W_r, W_ks):
    xb = H_K_prev.astype(jnp.bfloat16).reshape(K * N * 2, HALF // 32, 2, L)
    x2b = xb.transpose(0, 1, 3, 2).reshape(K * N * 2, HALF)
    npad = EPAD - E
    pad = jnp.concatenate(
        [jnp.zeros((K, 1, npad), jnp.int32),
         jnp.full((K, 1, npad), PAD_DST, jnp.int32)], axis=1)
    edges_flat = jnp.concatenate([edgelists, pad], axis=2).reshape(-1)
    z128 = jnp.zeros((NPAD, HALF), jnp.float32)
    z16 = jnp.zeros((NPAD, CW), jnp.float32)
    ones = jnp.ones((C, CW), jnp.float32)
    s_out, cnt_out = _get_sc_segment()(edges_flat, x2b, z128, z16, ones)
    a_w, b_w, c_w = _prep_weights(W_l, b_l.reshape(1, D), W_r, W_ks)
    out = _dense(H_K_prev, s_out[0], s_out[1], cnt_out[0], cnt_out[1],
                 a_w, b_w, c_w)
    return out[None]


# speedup vs baseline: 1.9054x; 1.9054x over previous
import functools

import jax
import jax.numpy as jnp
from jax import lax
from jax.experimental import pallas as pl
from jax.experimental.pallas import tpu as pltpu
from jax.experimental.pallas import tpu_sc as plsc

N = 10000
D = 256
K = 3
E = 160000

HALF = 128
NC = 2
NS = 16
L = 16
C = 128
NCH = 80
EPT = NCH * C
EPAD = EPT * NS
NPAD = 10240
RPT = NPAD // NS
CW = 16
PAD_DST = NPAD - 8


@functools.cache
def _get_sc_segment():
  mesh = plsc.VectorSubcoreMesh(core_axis_name="c", subcore_axis_name="s",
                                num_cores=NC, num_subcores=NS)

  @functools.partial(
      pl.kernel,
      out_type=(
          jax.ShapeDtypeStruct((NC, K, NPAD, HALF), jnp.float32),
          jax.ShapeDtypeStruct((NC, K, NPAD, CW), jnp.float32),
      ),
      mesh=mesh,
      compiler_params=pltpu.CompilerParams(use_tc_tiling_on_sc=False),
      scratch_types=[
          pltpu.VMEM((2, C), jnp.int32),
          pltpu.VMEM((2, C), jnp.int32),
          pltpu.VMEM((2, C, HALF), jnp.float32),
          pltpu.VMEM((C, CW), jnp.float32),
          pltpu.VMEM_SHARED((NPAD, HALF), jnp.float32),
          pltpu.VMEM_SHARED((NPAD, CW), jnp.float32),
          pltpu.SemaphoreType.DMA,
          pltpu.SemaphoreType.DMA,
          pltpu.SemaphoreType.DMA,
          pltpu.SemaphoreType.DMA,
      ],
  )
  def _sc_segment(edges_hbm, x2_hbm, z128_hbm, z16_hbm, ones_hbm,
                  s_out, cnt_out,
                  gidx_v, dst_v, rows_v, ones_v,
                  shared, cshared, gsem0, gsem1, ssem0, ssem1):
    cid = lax.axis_index("c")
    sid = lax.axis_index("s")
    r0 = pl.multiple_of(sid * RPT, 8)
    gsems = (gsem0, gsem1)
    ssems = (ssem0, ssem1)

    pltpu.sync_copy(ones_hbm, ones_v)

    for k in range(K):
      kbase = k * N
      src_base = (2 * k) * EPAD
      dst_base = (2 * k + 1) * EPAD

      pltpu.sync_copy(z128_hbm.at[pl.ds(r0, RPT)], shared.at[pl.ds(r0, RPT)])
      pltpu.sync_copy(z16_hbm.at[pl.ds(r0, RPT)], cshared.at[pl.ds(r0, RPT)])

      plsc.subcore_barrier()

      def fire(g, b):
        e0 = pl.multiple_of(sid * EPT + g * C, 8)
        pltpu.sync_copy(edges_hbm.at[pl.ds(src_base + e0, C)], gidx_v.at[b])
        pltpu.sync_copy(edges_hbm.at[pl.ds(dst_base + e0, C)], dst_v.at[b])
        for j in range(C // L):
          s16 = gidx_v[b, pl.ds(j * L, L)]
          gidx_v[b, pl.ds(j * L, L)] = (s16 + kbase) * 2 + cid
        pltpu.async_copy(x2_hbm.at[gidx_v.at[b]], rows_v.at[b], gsems[b])

      fire(0, 0)
      fire(1, 1)

      def body(i2, carry):
        for b in range(2):
          g = i2 * 2 + b
          pltpu.make_async_copy(x2_hbm.at[gidx_v.at[b]], rows_v.at[b],
                                gsems[b]).wait()
          pltpu.async_copy(rows_v.at[b], shared.at[dst_v.at[b]], ssems[b],
                           add=True)

          @pl.when(cid == b)
          def _():
            pltpu.sync_copy(ones_v, cshared.at[dst_v.at[b]], add=True)

          @pl.when(i2 < NCH // 2 - 1)
          def _():
            pltpu.make_async_copy(rows_v.at[b], shared.at[dst_v.at[b]],
                                  ssems[b]).wait()
            fire(g + 2, b)

        return carry

      lax.fori_loop(0, NCH // 2, body, 0)
      for b in range(2):
        pltpu.make_async_copy(rows_v.at[b], shared.at[dst_v.at[b]],
                              ssems[b]).wait()
      plsc.subcore_barrier()

      pltpu.sync_copy(shared.at[pl.ds(r0, RPT)],
                      s_out.at[cid, k, pl.ds(r0, RPT)])
      pltpu.sync_copy(cshared.at[pl.ds(r0, RPT)],
                      cnt_out.at[cid, k, pl.ds(r0, RPT)])

      plsc.subcore_barrier()

  return _sc_segment


def _prep_body(wl_ref, bl_ref, wr_ref, wk_ref, a_ref, b_ref, c_ref):
    for k in range(K):
        top = wk_ref[k, :D, :]
        bot = wk_ref[k, D:, :]
        a_ref[k] = top + jnp.dot(wr_ref[...], bot,
                                 preferred_element_type=jnp.float32)
        b_ref[k] = jnp.dot(wl_ref[...], bot,
                           preferred_element_type=jnp.float32)
    bsum = wk_ref[0, D:, :] + wk_ref[1, D:, :] + wk_ref[2, D:, :]
    c_ref[...] = jnp.dot(bl_ref[...], bsum,
                         preferred_element_type=jnp.float32)


_prep_weights = pl.pallas_call(
    _prep_body,
    out_shape=(
        jax.ShapeDtypeStruct((K, D, D), jnp.float32),
        jax.ShapeDtypeStruct((K, D, D), jnp.float32),
        jax.ShapeDtypeStruct((1, D), jnp.float32),
    ),
)

R_BLK = 2000


def _dense_body(x_ref, sl_ref, sr_ref, c0_ref, c1_ref, a_ref, b_ref, c_ref,
                o_ref):
    acc = jnp.broadcast_to(c_ref[...], (R_BLK, D))
    for k in range(K):
        cnt = c0_ref[k][:, 0:1] + c1_ref[k][:, 0:1]
        m = jnp.maximum(cnt, 1.0)
        agg_l = sl_ref[k] / m
        agg_r = sr_ref[k] / m
        acc = acc + jnp.dot(x_ref[k], a_ref[k],
                            preferred_element_type=jnp.float32)
        acc = acc + jnp.dot(agg_l, b_ref[k, :HALF, :],
                            preferred_element_type=jnp.float32)
        acc = acc + jnp.dot(agg_r, b_ref[k, HALF:, :],
                            preferred_element_type=jnp.float32)
    h = jnp.maximum(acc, 0.0)
    norm = jnp.sqrt(jnp.sum(h * h, axis=1, keepdims=True))
    o_ref[...] = h / jnp.maximum(norm, 1e-12)


_dense = pl.pallas_call(
    _dense_body,
    grid=(N // R_BLK,),
    in_specs=[
        pl.BlockSpec((K, R_BLK, D), lambda i: (0, i, 0)),
        pl.BlockSpec((K, R_BLK, HALF), lambda i: (0, i, 0)),
        pl.BlockSpec((K, R_BLK, HALF), lambda i: (0, i, 0)),
        pl.BlockSpec((K, R_BLK, CW), lambda i: (0, i, 0)),
        pl.BlockSpec((K, R_BLK, CW), lambda i: (0, i, 0)),
        pl.BlockSpec((K, D, D), lambda i: (0, 0, 0)),
        pl.BlockSpec((K, D, D), lambda i: (0, 0, 0)),
        pl.BlockSpec((1, D), lambda i: (0, 0)),
    ],
    out_specs=pl.BlockSpec((R_BLK, D), lambda i: (i, 0)),
    out_shape=jax.ShapeDtypeStruct((N, D), jnp.float32),
)


@jax.jit
def kernel(H_K_prev, edgelists, W_l, b_l, W_r, W_ks):
    x2 = H_K_prev.reshape(K * N * 2, HALF)
    npad = EPAD - E
    pad = jnp.concatenate(
        [jnp.zeros((K, 1, npad), jnp.int32),
         jnp.full((K, 1, npad), PAD_DST, jnp.int32)], axis=1)
    edges_flat = jnp.concatenate([edgelists, pad], axis=2).reshape(-1)
    z128 = jnp.zeros((NPAD, HALF), jnp.float32)
    z16 = jnp.zeros((NPAD, CW), jnp.float32)
    ones = jnp.ones((C, CW), jnp.float32)
    s_out, cnt_out = _get_sc_segment()(edges_flat, x2, z128, z16, ones)
    a_w, b_w, c_w = _prep_weights(W_l, b_l.reshape(1, D), W_r, W_ks)
    out = _dense(H_K_prev, s_out[0], s_out[1], cnt_out[0], cnt_out[1],
                 a_w, b_w, c_w)
    return out[None]


# speedup vs baseline: 1.9682x; 1.0329x over previous
import functools

import jax
import jax.numpy as jnp
from jax import lax
from jax.experimental import pallas as pl
from jax.experimental.pallas import tpu as pltpu
from jax.experimental.pallas import tpu_sc as plsc

N = 10000
D = 256
K = 3
E = 160000

HALF = 128
NC = 2
NS = 16
L = 16
C = 128
NCH = 80
EPT = NCH * C
EPAD = EPT * NS
NPAD = 10240
RPT = NPAD // NS
CW = 16
PAD_DST = NPAD - 8


@functools.cache
def _get_sc_segment():
  mesh = plsc.VectorSubcoreMesh(core_axis_name="c", subcore_axis_name="s",
                                num_cores=NC, num_subcores=NS)

  @functools.partial(
      pl.kernel,
      out_type=(
          jax.ShapeDtypeStruct((NC, K, NPAD, HALF), jnp.float32),
          jax.ShapeDtypeStruct((NC, K, NPAD, CW), jnp.float32),
      ),
      mesh=mesh,
      compiler_params=pltpu.CompilerParams(use_tc_tiling_on_sc=False),
      scratch_types=[
          pltpu.VMEM((4, C), jnp.int32),
          pltpu.VMEM((4, C), jnp.int32),
          pltpu.VMEM((2, C, HALF), jnp.float32),
          pltpu.VMEM((C, CW), jnp.float32),
          pltpu.VMEM_SHARED((NPAD, HALF), jnp.float32),
          pltpu.VMEM_SHARED((NPAD, CW), jnp.float32),
          pltpu.SemaphoreType.DMA,
          pltpu.SemaphoreType.DMA,
          pltpu.SemaphoreType.DMA,
          pltpu.SemaphoreType.DMA,
      ],
  )
  def _sc_segment(edges_hbm, x2_hbm, z128_hbm, z16_hbm, ones_hbm,
                  s_out, cnt_out,
                  gidx_v, dst_v, rows_v, ones_v,
                  shared, cshared, gsem0, gsem1, ssem0, ssem1):
    cid = lax.axis_index("c")
    sid = lax.axis_index("s")
    r0 = pl.multiple_of(sid * RPT, 8)
    gsems = (gsem0, gsem1)
    ssems = (ssem0, ssem1)

    pltpu.sync_copy(ones_hbm, ones_v)

    for k in range(K):
      kbase = k * N
      src_base = (2 * k) * EPAD
      dst_base = (2 * k + 1) * EPAD

      pltpu.sync_copy(z128_hbm.at[pl.ds(r0, RPT)], shared.at[pl.ds(r0, RPT)])
      pltpu.sync_copy(z16_hbm.at[pl.ds(r0, RPT)], cshared.at[pl.ds(r0, RPT)])

      plsc.subcore_barrier()

      def load_idx(g, ib):
        e0 = pl.multiple_of(sid * EPT + g * C, 8)
        pltpu.sync_copy(edges_hbm.at[pl.ds(src_base + e0, C)], gidx_v.at[ib])
        pltpu.sync_copy(edges_hbm.at[pl.ds(dst_base + e0, C)], dst_v.at[ib])
        for j in range(C // L):
          s16 = gidx_v[ib, pl.ds(j * L, L)]
          gidx_v[ib, pl.ds(j * L, L)] = (s16 + kbase) * 2 + cid

      def fire_gather(ib, b):
        pltpu.async_copy(x2_hbm.at[gidx_v.at[ib]], rows_v.at[b], gsems[b])

      load_idx(0, 0)
      load_idx(1, 1)
      fire_gather(0, 0)
      fire_gather(1, 1)

      def body(i4, carry):
        for u in range(4):
          g = i4 * 4 + u
          b = u % 2
          ib = u
          ib2 = (u + 2) % 4
          pltpu.make_async_copy(x2_hbm.at[gidx_v.at[ib]], rows_v.at[b],
                                gsems[b]).wait()
          pltpu.async_copy(rows_v.at[b], shared.at[dst_v.at[ib]], ssems[b],
                           add=True)

          @pl.when(cid == b)
          def _():
            pltpu.sync_copy(ones_v, cshared.at[dst_v.at[ib]], add=True)

          @pl.when(g + 2 < NCH)
          def _():
            load_idx(g + 2, ib2)
            pltpu.make_async_copy(rows_v.at[b], shared.at[dst_v.at[ib]],
                                  ssems[b]).wait()
            fire_gather(ib2, b)

        return carry

      lax.fori_loop(0, NCH // 4, body, 0)
      for b, ib in ((0, 2), (1, 3)):
        pltpu.make_async_copy(rows_v.at[b], shared.at[dst_v.at[ib]],
                              ssems[b]).wait()
      plsc.subcore_barrier()

      pltpu.sync_copy(shared.at[pl.ds(r0, RPT)],
                      s_out.at[cid, k, pl.ds(r0, RPT)])
      pltpu.sync_copy(cshared.at[pl.ds(r0, RPT)],
                      cnt_out.at[cid, k, pl.ds(r0, RPT)])

      plsc.subcore_barrier()

  return _sc_segment


def _prep_body(wl_ref, bl_ref, wr_ref, wk_ref, a_ref, b_ref, c_ref):
    for k in range(K):
        top = wk_ref[k, :D, :]
        bot = wk_ref[k, D:, :]
        a_ref[k] = top + jnp.dot(wr_ref[...], bot,
                                 preferred_element_type=jnp.float32)
        b_ref[k] = jnp.dot(wl_ref[...], bot,
                           preferred_element_type=jnp.float32)
    bsum = wk_ref[0, D:, :] + wk_ref[1, D:, :] + wk_ref[2, D:, :]
    c_ref[...] = jnp.dot(bl_ref[...], bsum,
                         preferred_element_type=jnp.float32)


_prep_weights = pl.pallas_call(
    _prep_body,
    out_shape=(
        jax.ShapeDtypeStruct((K, D, D), jnp.float32),
        jax.ShapeDtypeStruct((K, D, D), jnp.float32),
        jax.ShapeDtypeStruct((1, D), jnp.float32),
    ),
)

R_BLK = 2000


def _dense_body(x_ref, sl_ref, sr_ref, c0_ref, c1_ref, a_ref, b_ref, c_ref,
                o_ref):
    acc = jnp.broadcast_to(c_ref[...], (R_BLK, D))
    for k in range(K):
        cnt = c0_ref[k][:, 0:1] + c1_ref[k][:, 0:1]
        m = jnp.maximum(cnt, 1.0)
        agg_l = sl_ref[k] / m
        agg_r = sr_ref[k] / m
        acc = acc + jnp.dot(x_ref[k], a_ref[k],
                            preferred_element_type=jnp.float32)
        acc = acc + jnp.dot(agg_l, b_ref[k, :HALF, :],
                            preferred_element_type=jnp.float32)
        acc = acc + jnp.dot(agg_r, b_ref[k, HALF:, :],
                            preferred_element_type=jnp.float32)
    h = jnp.maximum(acc, 0.0)
    norm = jnp.sqrt(jnp.sum(h * h, axis=1, keepdims=True))
    o_ref[...] = h / jnp.maximum(norm, 1e-12)


_dense = pl.pallas_call(
    _dense_body,
    grid=(N // R_BLK,),
    in_specs=[
        pl.BlockSpec((K, R_BLK, D), lambda i: (0, i, 0)),
        pl.BlockSpec((K, R_BLK, HALF), lambda i: (0, i, 0)),
        pl.BlockSpec((K, R_BLK, HALF), lambda i: (0, i, 0)),
        pl.BlockSpec((K, R_BLK, CW), lambda i: (0, i, 0)),
        pl.BlockSpec((K, R_BLK, CW), lambda i: (0, i, 0)),
        pl.BlockSpec((K, D, D), lambda i: (0, 0, 0)),
        pl.BlockSpec((K, D, D), lambda i: (0, 0, 0)),
        pl.BlockSpec((1, D), lambda i: (0, 0)),
    ],
    out_specs=pl.BlockSpec((R_BLK, D), lambda i: (i, 0)),
    out_shape=jax.ShapeDtypeStruct((N, D), jnp.float32),
)


@jax.jit
def kernel(H_K_prev, edgelists, W_l, b_l, W_r, W_ks):
    x2 = H_K_prev.reshape(K * N * 2, HALF)
    npad = EPAD - E
    pad = jnp.concatenate(
        [jnp.zeros((K, 1, npad), jnp.int32),
         jnp.full((K, 1, npad), PAD_DST, jnp.int32)], axis=1)
    edges_flat = jnp.concatenate([edgelists, pad], axis=2).reshape(-1)
    z128 = jnp.zeros((NPAD, HALF), jnp.float32)
    z16 = jnp.zeros((NPAD, CW), jnp.float32)
    ones = jnp.ones((C, CW), jnp.float32)
    s_out, cnt_out = _get_sc_segment()(edges_flat, x2, z128, z16, ones)
    a_w, b_w, c_w = _prep_weights(W_l, b_l.reshape(1, D), W_r, W_ks)
    out = _dense(H_K_prev, s_out[0], s_out[1], cnt_out[0], cnt_out[1],
                 a_w, b_w, c_w)
    return out[None]
